# baseline (speedup 1.0000x reference)
import jax
import jax.numpy as jnp
from jax import lax
from jax.experimental import pallas as pl
from jax.experimental.pallas import tpu as pltpu
from jax.experimental.pallas import tpu_sc as plsc

D_MODEL = 64
SCALE = 8.0

NC = 2
NS = 16
NW = NC * NS

BB = 128
LANES = 16


def _body(xt_hbm, t128_hbm, out_hbm, idx_v, rows0, rows1, tr0, tr1,
          gsem0, gsem1, osem0, osem1):
    seq = xt_hbm.shape[0]
    wid = lax.axis_index("s") * NC + lax.axis_index("c")
    bcol = pl.multiple_of(wid * BB, BB)

    pltpu.sync_copy(xt_hbm.at[:, pl.ds(bcol, BB)], idx_v)

    rows = (rows0, rows1)
    trs = (tr0, tr1)
    gsems = (gsem0, gsem1)
    osems = (osem0, osem1)

    def gather_copy(s, buf, sem):
        return pltpu.make_async_copy(t128_hbm.at[idx_v.at[s]], buf, sem)

    def out_copy(s, buf, sem):
        return pltpu.make_async_copy(
            buf, out_hbm.at[s, :, pl.ds(bcol, BB)], sem)

    lanes_iota = lax.iota(jnp.int32, LANES)

    def transpose_scale(rbuf, tbuf):
        def trow(f, carry):
            for g in range(BB // LANES):
                tok_idx = lanes_iota + g * LANES
                feat_idx = jnp.full((LANES,), 0, jnp.int32) + f
                vals = plsc.load_gather(rbuf, [tok_idx, feat_idx])
                tbuf[f, pl.ds(g * LANES, LANES)] = vals * SCALE
            return carry
        lax.fori_loop(0, D_MODEL, trow, 0, unroll=2)

    gather_copy(0, rows[0], gsems[0]).start()

    def step(su, carry):
        for par in range(2):
            s = 2 * su + par
            nbuf = 1 - par

            @pl.when(s + 1 < seq)
            def _():
                gather_copy(s + 1, rows[nbuf], gsems[nbuf]).start()

            gather_copy(s, rows[par], gsems[par]).wait()

            @pl.when(s >= 2)
            def _():
                out_copy(s - 2, trs[par], osems[par]).wait()

            transpose_scale(rows[par], trs[par])
            out_copy(s, trs[par], osems[par]).start()
        return carry

    lax.fori_loop(0, seq // 2, step, 0)
    out_copy(seq - 2, trs[0], osems[0]).wait()
    out_copy(seq - 1, trs[1], osems[1]).wait()


def kernel(x, table):
    b_total, seq = x.shape
    xt = x.T.astype(jnp.int32)
    t128 = jnp.pad(table, ((0, 0), (0, 64)))
    mesh = plsc.VectorSubcoreMesh(core_axis_name="c", subcore_axis_name="s")
    out_phys = pl.kernel(
        _body,
        mesh=mesh,
        compiler_params=pltpu.CompilerParams(
            use_tc_tiling_on_sc=True, needs_layout_passes=False),
        out_type=jax.ShapeDtypeStruct((seq, D_MODEL, b_total), jnp.float32),
        scratch_types=[
            pltpu.VMEM((seq, BB), jnp.int32),
            pltpu.VMEM((BB, 128), jnp.float32),
            pltpu.VMEM((BB, 128), jnp.float32),
            pltpu.VMEM((D_MODEL, BB), jnp.float32),
            pltpu.VMEM((D_MODEL, BB), jnp.float32),
            pltpu.SemaphoreType.DMA,
            pltpu.SemaphoreType.DMA,
            pltpu.SemaphoreType.DMA,
            pltpu.SemaphoreType.DMA,
        ],
    )(xt, t128)
    return out_phys.transpose(2, 0, 1)


# speedup vs baseline: 1.1348x; 1.1348x over previous
import jax
import jax.numpy as jnp
from jax import lax
from jax.experimental import pallas as pl
from jax.experimental.pallas import tpu as pltpu
from jax.experimental.pallas import tpu_sc as plsc

D_MODEL = 64
SCALE = 8.0

NC = 2
NS = 16
NW = NC * NS

BB = 128
LANES = 16
NGB = 4
NTB = 2


def _body(xt_hbm, t128_hbm, out_hbm, idx_v, r0, r1, r2, r3, t0, t1,
          g0, g1, g2, g3, o0, o1):
    seq = xt_hbm.shape[0]
    wid = lax.axis_index("s") * NC + lax.axis_index("c")
    bcol = pl.multiple_of(wid * BB, BB)

    pltpu.sync_copy(xt_hbm.at[:, pl.ds(bcol, BB)], idx_v)

    rows = (r0, r1, r2, r3)
    trs = (t0, t1)
    gsems = (g0, g1, g2, g3)
    osems = (o0, o1)

    def gather_copy(s, k):
        return pltpu.make_async_copy(
            t128_hbm.at[idx_v.at[s]], rows[k], gsems[k])

    def out_copy(s, k):
        return pltpu.make_async_copy(
            trs[k], out_hbm.at[s, :, pl.ds(bcol, BB)], osems[k])

    iota = lax.iota(jnp.int32, LANES)
    fidx = [iota + j * LANES for j in range(D_MODEL // LANES)]

    def transpose_scale(rbuf, tbuf):
        def trow(b, carry):
            bvec = jnp.full((LANES,), 0, jnp.int32) + b
            for j in range(D_MODEL // LANES):
                vals = rbuf[b, pl.ds(j * LANES, LANES)] * SCALE
                plsc.store_scatter(tbuf, [fidx[j], bvec], vals)
            return carry
        lax.fori_loop(0, BB, trow, 0, unroll=4)

    for k in range(NGB):
        gather_copy(k, k).start()

    def step(su, carry):
        for par in range(NGB):
            s = NGB * su + par
            gather_copy(s, par).wait()

            @pl.when(s >= NTB)
            def _():
                out_copy(s - NTB, par % NTB).wait()

            transpose_scale(rows[par], trs[par % NTB])

            @pl.when(s + NGB < seq)
            def _():
                gather_copy(s + NGB, par).start()

            out_copy(s, par % NTB).start()
        return carry

    lax.fori_loop(0, seq // NGB, step, 0)
    out_copy(seq - 2, 0).wait()
    out_copy(seq - 1, 1).wait()


def kernel(x, table):
    b_total, seq = x.shape
    xt = x.T.astype(jnp.int32)
    t128 = jnp.pad(table, ((0, 0), (0, 64)))
    mesh = plsc.VectorSubcoreMesh(core_axis_name="c", subcore_axis_name="s")
    out_phys = pl.kernel(
        _body,
        mesh=mesh,
        compiler_params=pltpu.CompilerParams(
            use_tc_tiling_on_sc=True, needs_layout_passes=False),
        out_type=jax.ShapeDtypeStruct((seq, D_MODEL, b_total), jnp.float32),
        scratch_types=[
            pltpu.VMEM((seq, BB), jnp.int32),
            pltpu.VMEM((BB, 128), jnp.float32),
            pltpu.VMEM((BB, 128), jnp.float32),
            pltpu.VMEM((BB, 128), jnp.float32),
            pltpu.VMEM((BB, 128), jnp.float32),
            pltpu.VMEM((D_MODEL, BB), jnp.float32),
            pltpu.VMEM((D_MODEL, BB), jnp.float32),
            pltpu.SemaphoreType.DMA,
            pltpu.SemaphoreType.DMA,
            pltpu.SemaphoreType.DMA,
            pltpu.SemaphoreType.DMA,
            pltpu.SemaphoreType.DMA,
            pltpu.SemaphoreType.DMA,
        ],
    )(xt, t128)
    return out_phys.transpose(2, 0, 1)


# speedup vs baseline: 1.5101x; 1.3308x over previous
import jax
import jax.numpy as jnp
from jax import lax
from jax.experimental import pallas as pl
from jax.experimental.pallas import tpu as pltpu
from jax.experimental.pallas import tpu_sc as plsc

D_MODEL = 64
SCALE = 8.0

NC = 2
NS = 16
NW = NC * NS

BB = 128
LANES = 16
NGB = 4
NTB = 2


def _body(xt_hbm, t128_hbm, out_hbm, idx_v, r0, r1, r2, r3, t0, t1,
          g0, g1, g2, g3, o0, o1):
    seq = xt_hbm.shape[0]
    wid = lax.axis_index("s") * NC + lax.axis_index("c")
    bcol = pl.multiple_of(wid * BB, BB)

    pltpu.sync_copy(xt_hbm.at[:, pl.ds(bcol, BB)], idx_v)

    rows = (r0, r1, r2, r3)
    trs = (t0, t1)
    gsems = (g0, g1, g2, g3)
    osems = (o0, o1)

    def gather_copy(s, k):
        return pltpu.make_async_copy(
            t128_hbm.at[idx_v.at[s]], rows[k], gsems[k])

    def out_copy(s, k):
        return pltpu.make_async_copy(
            trs[k], out_hbm.at[s, :, pl.ds(bcol, BB)], osems[k])

    iota = lax.iota(jnp.int32, LANES)
    fidx = [iota + j * LANES for j in range(D_MODEL // LANES)]

    def transpose_scale(rbuf, tbuf):
        @plsc.parallel_loop(0, BB, 1, unroll=8)
        def trow(b):
            bvec = jnp.full((LANES,), 0, jnp.int32) + b
            for j in range(D_MODEL // LANES):
                vals = rbuf[b, pl.ds(j * LANES, LANES)] * SCALE
                plsc.store_scatter(tbuf, [fidx[j], bvec], vals)

    for k in range(NGB):
        gather_copy(k, k).start()

    def step(su, carry):
        for par in range(NGB):
            s = NGB * su + par
            gather_copy(s, par).wait()

            @pl.when(s >= NTB)
            def _():
                out_copy(s - NTB, par % NTB).wait()

            transpose_scale(rows[par], trs[par % NTB])

            @pl.when(s + NGB < seq)
            def _():
                gather_copy(s + NGB, par).start()

            out_copy(s, par % NTB).start()
        return carry

    lax.fori_loop(0, seq // NGB, step, 0)
    out_copy(seq - 2, 0).wait()
    out_copy(seq - 1, 1).wait()


def kernel(x, table):
    b_total, seq = x.shape
    xt = x.T.astype(jnp.int32)
    t128 = jnp.pad(table, ((0, 0), (0, 64)))
    mesh = plsc.VectorSubcoreMesh(core_axis_name="c", subcore_axis_name="s")
    out_phys = pl.kernel(
        _body,
        mesh=mesh,
        compiler_params=pltpu.CompilerParams(
            use_tc_tiling_on_sc=True, needs_layout_passes=False),
        out_type=jax.ShapeDtypeStruct((seq, D_MODEL, b_total), jnp.float32),
        scratch_types=[
            pltpu.VMEM((seq, BB), jnp.int32),
            pltpu.VMEM((BB, 128), jnp.float32),
            pltpu.VMEM((BB, 128), jnp.float32),
            pltpu.VMEM((BB, 128), jnp.float32),
            pltpu.VMEM((BB, 128), jnp.float32),
            pltpu.VMEM((D_MODEL, BB), jnp.float32),
            pltpu.VMEM((D_MODEL, BB), jnp.float32),
            pltpu.SemaphoreType.DMA,
            pltpu.SemaphoreType.DMA,
            pltpu.SemaphoreType.DMA,
            pltpu.SemaphoreType.DMA,
            pltpu.SemaphoreType.DMA,
            pltpu.SemaphoreType.DMA,
        ],
    )(xt, t128)
    return out_phys.transpose(2, 0, 1)
